# Initial kernel scaffold; baseline (speedup 1.0000x reference)
import jax
import jax.numpy as jnp
from jax.experimental import pallas as pl


def kernel(x, pos_table):
    raise NotImplementedError("write your pallas kernel here")


# speedup vs baseline: 3.1747x; 3.1747x over previous
import jax
import jax.numpy as jnp
from jax.experimental import pallas as pl

_BLK = 1024


def _add_block(x_ref, p_ref, o_ref):
    o_ref[...] = x_ref[...] + p_ref[...]


def kernel(x, pos_table):
    batch, seq_len, d_model = x.shape
    nblk = seq_len // _BLK
    return pl.pallas_call(
        _add_block,
        grid=(nblk, batch),
        in_specs=[
            pl.BlockSpec((1, _BLK, d_model), lambda s, b: (b, s, 0)),
            pl.BlockSpec((_BLK, d_model), lambda s, b: (s, 0)),
        ],
        out_specs=pl.BlockSpec((1, _BLK, d_model), lambda s, b: (b, s, 0)),
        out_shape=jax.ShapeDtypeStruct(x.shape, x.dtype),
    )(x, pos_table)


# speedup vs baseline: 3.3139x; 1.0438x over previous
import jax
import jax.numpy as jnp
from jax.experimental import pallas as pl

_BLK = 2048


def _add_block(x_ref, p_ref, o_ref):
    o_ref[...] = x_ref[...] + p_ref[...]


def kernel(x, pos_table):
    batch, seq_len, d_model = x.shape
    nblk = seq_len // _BLK
    return pl.pallas_call(
        _add_block,
        grid=(nblk, batch),
        in_specs=[
            pl.BlockSpec((1, _BLK, d_model), lambda s, b: (b, s, 0)),
            pl.BlockSpec((_BLK, d_model), lambda s, b: (s, 0)),
        ],
        out_specs=pl.BlockSpec((1, _BLK, d_model), lambda s, b: (b, s, 0)),
        out_shape=jax.ShapeDtypeStruct(x.shape, x.dtype),
    )(x, pos_table)
